# baseline (speedup 1.0000x reference)
import functools

import jax
import jax.numpy as jnp
from jax import lax
from jax.experimental import pallas as pl
from jax.experimental.pallas import tpu as pltpu
from jax.experimental.pallas import tpu_sc as plsc

_S, _N = 60, 4096
_SN = _S * _N
_NARGS = 16
_D = 256
_DW = _D // 2
_AE = 64
_NCMD = 6
_NPAIR = 7
_NG = 9
_A_ROWS = _NCMD * 256 + 256
_B_ROWS = _NPAIR * 65536

_NC, _NS = 2, 16
_NW = _NC * _NS
_P = 80
_H = _P // 2
_PER_W = _SN // _NW
_NBLK = _PER_W // _P
_NB_TOT = _SN // _P


def _stage1_body(at1_ref, w_ref, cmd_ref, b_ref, ta_ref, tmid_ref):
  at1 = at1_ref[...]
  t0 = jnp.dot(at1, w_ref[pl.ds(0, _AE), :],
               preferred_element_type=jnp.float32)
  t0 = t0 + b_ref[...]
  for c in range(_NCMD):
    ta_ref[pl.ds(c * 256, 256), :] = (
        t0 + cmd_ref[pl.ds(c, 1), :]).astype(jnp.bfloat16)
  t15 = jnp.dot(at1, w_ref[pl.ds(15 * _AE, _AE), :],
                preferred_element_type=jnp.float32)
  ta_ref[pl.ds(_NCMD * 256, 256), :] = t15.astype(jnp.bfloat16)
  for k in range(1, 15):
    tk = jnp.dot(at1, w_ref[pl.ds(k * _AE, _AE), :],
                 preferred_element_type=jnp.float32)
    tmid_ref[k - 1] = tk


def _stage1(arg_table, W, cmd_table, b):
  at1 = arg_table[1:257]
  cmdp = jnp.pad(cmd_table, ((0, 2), (0, 0)))
  return pl.pallas_call(
      _stage1_body,
      out_shape=(jax.ShapeDtypeStruct((_A_ROWS, _D), jnp.bfloat16),
                 jax.ShapeDtypeStruct((14, 256, _D), jnp.float32)),
  )(at1, W, cmdp, b.reshape(1, _D))


def _stage2_body(rowt_ref, colt_ref, out_ref):
  col = colt_ref[0]
  for a_loc in range(32):
    out_ref[pl.ds(a_loc * 256, 256), :] = (
        col + rowt_ref[0, pl.ds(a_loc, 1), :]).astype(jnp.bfloat16)


def _stage2(tmid):
  return pl.pallas_call(
      _stage2_body,
      grid=(_NPAIR, 8),
      in_specs=[
          pl.BlockSpec((1, 32, _D), lambda j, i: (2 * j, i, 0)),
          pl.BlockSpec((1, 256, _D), lambda j, i: (2 * j + 1, 0, 0)),
      ],
      out_specs=pl.BlockSpec((32 * 256, _D), lambda j, i: (j * 8 + i, 0)),
      out_shape=jax.ShapeDtypeStruct((_B_ROWS, _D), jnp.bfloat16),
  )(tmid, tmid)


def _pack_words(t_bf16):
  rows = t_bf16.shape[0]
  return lax.bitcast_convert_type(t_bf16.reshape(rows, _DW, 2), jnp.int32)


def _build_idx(raw_v, idx_v):
  for t in range(_P // 16):
    sl = pl.ds(t * 16, 16)
    idx_v[0, sl] = raw_v[0, sl] * 256 + raw_v[1, sl]
    for g in range(1, 1 + _NPAIR):
      idx_v[g, sl] = (raw_v[2 * g, sl] * 256 + raw_v[2 * g + 1, sl]
                      + (g - 1) * 65536)
    idx_v[_NG - 1, sl] = raw_v[16, sl] + _NCMD * 256


def _fold(sts, out_v):
  @plsc.parallel_loop(0, _H, unroll=4)
  def row(r):
    for t in range(_DW // 16):
      sl = pl.ds(t * 16, 16)
      ws = [sts[g][r, sl] for g in range(_NG)]
      los = [lax.bitcast_convert_type(lax.shift_left(w, jnp.int32(16)),
                                      jnp.float32) for w in ws]
      his = [lax.bitcast_convert_type(lax.bitwise_and(w, jnp.int32(-65536)),
                                      jnp.float32) for w in ws]
      while len(los) > 1:
        los = [los[i] + los[i + 1] for i in range(0, len(los) - 1, 2)] \
            + ([los[-1]] if len(los) % 2 else [])
        his = [his[i] + his[i + 1] for i in range(0, len(his) - 1, 2)] \
            + ([his[-1]] if len(his) % 2 else [])
      out_v[r, sl] = los[0]
      out_v[r, pl.ds(_DW + t * 16, 16)] = his[0]


def _sc_body(slab_hbm, ta_hbm, tb_hbm, out_hbm, raw_a, raw_b, idx_a, idx_b,
             out0_v, out1_v, *rest):
  st0 = rest[:_NG]
  st1 = rest[_NG:2 * _NG]
  sem_s0, sem_s1, sem_o0, sem_o1 = rest[2 * _NG:]
  wid = lax.axis_index("s") * _NC + lax.axis_index("c")
  tabs = [ta_hbm] + [tb_hbm] * _NPAIR + [ta_hbm]
  wbase = wid * _NBLK

  def fire9(idx_v, h, sts, sem):
    return [pltpu.async_copy(
        tabs[g].at[idx_v.at[g, pl.ds(h * _H, _H)]], sts[g], sem)
        for g in range(_NG)]

  def drain9(sts, sem):
    for g in range(_NG):
      pltpu.make_async_copy(tb_hbm.at[pl.ds(0, _H)], sts[g], sem).wait()

  def drain_out(out_v, sem):
    pltpu.make_async_copy(out_v, out_hbm.at[pl.ds(0, _H)], sem).wait()

  pltpu.async_copy(out_hbm.at[pl.ds(wbase * _P, _H)], out0_v, sem_o0)
  pltpu.async_copy(out_hbm.at[pl.ds(wbase * _P, _H)], out1_v, sem_o1)
  pltpu.sync_copy(slab_hbm.at[wbase], raw_a)
  _build_idx(raw_a, idx_a)
  fire9(idx_a, 0, st0, sem_s0)

  def body(j2, carry):
    b0 = wbase + 2 * j2
    b1 = b0 + 1
    b2 = wbase + jnp.minimum(2 * j2 + 2, _NBLK - 1)
    d_s1 = fire9(idx_a, 1, st1, sem_s1)
    drain9(st0, sem_s0)
    drain_out(out0_v, sem_o0)
    _fold(st0, out0_v)
    d_o0 = pltpu.async_copy(out0_v, out_hbm.at[pl.ds(b0 * _P, _H)], sem_o0)
    pltpu.sync_copy(slab_hbm.at[b1], raw_b)
    _build_idx(raw_b, idx_b)
    d_s0 = fire9(idx_b, 0, st0, sem_s0)
    for d in d_s1:
      d.wait()
    drain_out(out1_v, sem_o1)
    _fold(st1, out1_v)
    d_o1 = pltpu.async_copy(
        out1_v, out_hbm.at[pl.ds(b0 * _P + _H, _H)], sem_o1)
    for d in d_s0:
      d.wait()
    d_s1b = fire9(idx_b, 1, st1, sem_s1)
    d_o0.wait()
    _fold(st0, out0_v)
    pltpu.async_copy(out0_v, out_hbm.at[pl.ds(b1 * _P, _H)], sem_o0)
    pltpu.sync_copy(slab_hbm.at[b2], raw_a)
    _build_idx(raw_a, idx_a)
    fire9(idx_a, 0, st0, sem_s0)
    for d in d_s1b:
      d.wait()
    d_o1.wait()
    _fold(st1, out1_v)
    pltpu.async_copy(out1_v, out_hbm.at[pl.ds(b1 * _P + _H, _H)], sem_o1)
    return carry

  lax.fori_loop(0, _NBLK // 2, body, 0)
  drain9(st0, sem_s0)
  drain_out(out0_v, sem_o0)
  drain_out(out1_v, sem_o1)


def _sc_gather_sum(slab, table_a, table_b):
  mesh = plsc.VectorSubcoreMesh(core_axis_name="c", subcore_axis_name="s")
  f = pl.kernel(
      _sc_body,
      out_type=jax.ShapeDtypeStruct((_SN, _D), jnp.float32),
      mesh=mesh,
      scratch_types=[
          pltpu.VMEM((_NARGS + 1, _P), jnp.int32),
          pltpu.VMEM((_NARGS + 1, _P), jnp.int32),
          pltpu.VMEM((_NG, _P), jnp.int32),
          pltpu.VMEM((_NG, _P), jnp.int32),
          pltpu.VMEM((_H, _D), jnp.float32),
          pltpu.VMEM((_H, _D), jnp.float32),
      ] + [pltpu.VMEM((_H, _DW), jnp.int32) for _ in range(2 * _NG)]
        + [pltpu.SemaphoreType.DMA for _ in range(4)],
  )
  return f(slab, table_a, table_b)


def kernel(commands, args, cmd_table, arg_table, W, b):
  sigma = jnp.stack(
      [jnp.arange(_DW, dtype=jnp.int32),
       jnp.arange(_DW, dtype=jnp.int32) + _DW], axis=1).reshape(_D)
  table_a, tmid = _stage1(arg_table, W[:, sigma], cmd_table[:, sigma], b[sigma])
  table_b = _stage2(tmid)
  flat = jnp.concatenate(
      [commands.reshape(_SN, 1), args.reshape(_SN, _NARGS)], axis=1)
  slab = flat.reshape(_NB_TOT, _P, _NARGS + 1).swapaxes(1, 2)
  out = _sc_gather_sum(slab, _pack_words(table_a), _pack_words(table_b))
  return out.reshape(_S, _N, _D)


# speedup vs baseline: 1.8759x; 1.8759x over previous
import functools

import jax
import jax.numpy as jnp
from jax import lax
from jax.experimental import pallas as pl
from jax.experimental.pallas import tpu as pltpu
from jax.experimental.pallas import tpu_sc as plsc

_S, _N = 60, 4096
_SN = _S * _N
_NARGS = 16
_D = 256
_AE = 64
_NCMD = 6
_NPAIR = 7
_NG = 9
_A_ROWS = _NCMD * 256 + 256
_B_ROWS = _NPAIR * 65536

_NC, _NS = 2, 16
_NW = _NC * _NS
_P = 96
_NST = 2
_PER_W = _SN // _NW
_NBLK = _PER_W // _P
_NB_TOT = _SN // _P


def _stage1_body(at1_ref, w_ref, cmd_ref, b_ref, ta_ref, tmid_ref):
  at1 = at1_ref[...]
  t0 = jnp.dot(at1, w_ref[pl.ds(0, _AE), :],
               preferred_element_type=jnp.float32)
  t0 = t0 + b_ref[...]
  for c in range(_NCMD):
    ta_ref[pl.ds(c * 256, 256), :] = t0 + cmd_ref[pl.ds(c, 1), :]
  t15 = jnp.dot(at1, w_ref[pl.ds(15 * _AE, _AE), :],
                preferred_element_type=jnp.float32)
  ta_ref[pl.ds(_NCMD * 256, 256), :] = t15
  for k in range(1, 15):
    tk = jnp.dot(at1, w_ref[pl.ds(k * _AE, _AE), :],
                 preferred_element_type=jnp.float32)
    tmid_ref[k - 1] = tk


def _stage1(arg_table, W, cmd_table, b):
  at1 = arg_table[1:257]
  cmdp = jnp.pad(cmd_table, ((0, 2), (0, 0)))
  return pl.pallas_call(
      _stage1_body,
      out_shape=(jax.ShapeDtypeStruct((_A_ROWS, _D), jnp.float32),
                 jax.ShapeDtypeStruct((14, 256, _D), jnp.float32)),
  )(at1, W, cmdp, b.reshape(1, _D))


def _stage2_body(rowt_ref, colt_ref, out_ref):
  col = colt_ref[0]
  for a_loc in range(32):
    out_ref[pl.ds(a_loc * 256, 256), :] = col + rowt_ref[0, pl.ds(a_loc, 1), :]


def _stage2(tmid):
  return pl.pallas_call(
      _stage2_body,
      grid=(_NPAIR, 8),
      in_specs=[
          pl.BlockSpec((1, 32, _D), lambda j, i: (2 * j, i, 0)),
          pl.BlockSpec((1, 256, _D), lambda j, i: (2 * j + 1, 0, 0)),
      ],
      out_specs=pl.BlockSpec((32 * 256, _D), lambda j, i: (j * 8 + i, 0)),
      out_shape=jax.ShapeDtypeStruct((_B_ROWS, _D), jnp.float32),
  )(tmid, tmid)


def _accumulate(acc_v, st_v):
  def row(r, carry):
    for t in range(_D // 16):
      sl = pl.ds(t * 16, 16)
      plsc.addupdate(acc_v.at[r, sl], st_v[r, sl])
    return carry
  lax.fori_loop(0, _P, row, 0)


def _sc_body(slab_hbm, ta_hbm, tb_hbm, out_hbm, raw_a, raw_b, idx_a, idx_b,
             acc_a, acc_b, st0_v, st1_v, sem_ga, sem_gb, sem_oa, sem_ob,
             sem_s0, sem_s1):
  sts = (st0_v, st1_v)
  sems = (sem_s0, sem_s1)
  wid = lax.axis_index("s") * _NC + lax.axis_index("c")
  tabs = [ta_hbm] + [tb_hbm] * _NPAIR + [ta_hbm]
  wbase = wid * _NBLK

  def build_idx(raw_v, idx_v):
    for t in range(_P // 16):
      sl = pl.ds(t * 16, 16)
      idx_v[0, sl] = raw_v[0, sl] * 256 + raw_v[1, sl]
      for g in range(1, 1 + _NPAIR):
        idx_v[g, sl] = (raw_v[2 * g, sl] * 256 + raw_v[2 * g + 1, sl]
                        + (g - 1) * 65536)
      idx_v[_NG - 1, sl] = raw_v[16, sl] + _NCMD * 256

  def drain_acc(acc_v, sem):
    pltpu.make_async_copy(ta_hbm.at[pl.ds(0, _P)], acc_v, sem).wait()

  def phase(idx_me, acc_me, sem_me, prefetch, tail):
    descs = {1: pltpu.async_copy(tabs[1].at[idx_me.at[1]], sts[1], sems[1])}
    prefetch()
    drain_acc(acc_me, sem_me)
    for g in range(1, _NG):
      nxt = g + _NST - 1
      if nxt < _NG:
        descs[nxt] = pltpu.async_copy(
            tabs[nxt].at[idx_me.at[nxt]], sts[nxt % _NST], sems[nxt % _NST])
      descs[g].wait()
      _accumulate(acc_me, sts[g % _NST])
    return tail()

  def fire0(idx_v, acc_v, sem):
    return pltpu.async_copy(ta_hbm.at[idx_v.at[0]], acc_v, sem)

  pltpu.sync_copy(slab_hbm.at[wbase], raw_a)
  build_idx(raw_a, idx_a)
  fire0(idx_a, acc_a, sem_ga)

  def body(j2, carry):
    gb0 = wbase + 2 * j2
    gb1 = gb0 + 1
    gb2 = wbase + jnp.minimum(2 * j2 + 2, _NBLK - 1)

    def prefetch0():
      pltpu.sync_copy(slab_hbm.at[gb1], raw_b)
      build_idx(raw_b, idx_b)

    def tail0():
      d_oa = pltpu.async_copy(acc_a, out_hbm.at[pl.ds(gb0 * _P, _P)], sem_oa)
      @pl.when(j2 > 0)
      def _():
        pltpu.make_async_copy(acc_b, out_hbm.at[pl.ds(0, _P)], sem_ob).wait()
      fire0(idx_b, acc_b, sem_gb)
      return d_oa

    d_oa = phase(idx_a, acc_a, sem_ga, prefetch0, tail0)

    def prefetch1():
      pltpu.sync_copy(slab_hbm.at[gb2], raw_a)
      build_idx(raw_a, idx_a)

    def tail1():
      pltpu.async_copy(acc_b, out_hbm.at[pl.ds(gb1 * _P, _P)], sem_ob)
      d_oa.wait()
      fire0(idx_a, acc_a, sem_ga)
      return 0

    phase(idx_b, acc_b, sem_gb, prefetch1, tail1)
    return carry

  lax.fori_loop(0, _NBLK // 2, body, 0)
  drain_acc(acc_a, sem_ga)
  pltpu.make_async_copy(acc_b, out_hbm.at[pl.ds(0, _P)], sem_ob).wait()


def _sc_gather_sum(slab, table_a, table_b):
  mesh = plsc.VectorSubcoreMesh(core_axis_name="c", subcore_axis_name="s")
  f = pl.kernel(
      _sc_body,
      out_type=jax.ShapeDtypeStruct((_SN, _D), jnp.float32),
      mesh=mesh,
      scratch_types=[
          pltpu.VMEM((_NARGS + 1, _P), jnp.int32),
          pltpu.VMEM((_NARGS + 1, _P), jnp.int32),
          pltpu.VMEM((_NG, _P), jnp.int32),
          pltpu.VMEM((_NG, _P), jnp.int32),
          pltpu.VMEM((_P, _D), jnp.float32),
          pltpu.VMEM((_P, _D), jnp.float32),
          pltpu.VMEM((_P, _D), jnp.float32),
          pltpu.VMEM((_P, _D), jnp.float32),
      ] + [pltpu.SemaphoreType.DMA for _ in range(6)],
  )
  return f(slab, table_a, table_b)


def kernel(commands, args, cmd_table, arg_table, W, b):
  table_a, tmid = _stage1(arg_table, W, cmd_table, b)
  table_b = _stage2(tmid)
  flat = jnp.concatenate(
      [commands.reshape(_SN, 1), args.reshape(_SN, _NARGS)], axis=1)
  slab = flat.reshape(_NB_TOT, _P, _NARGS + 1).swapaxes(1, 2)
  out = _sc_gather_sum(slab, table_a, table_b)
  return out.reshape(_S, _N, _D)
